# Initial kernel scaffold; baseline (speedup 1.0000x reference)
import jax
import jax.numpy as jnp
from jax.experimental import pallas as pl


def kernel(entity_re, entity_im, relation_re, relation_im, head, tail, relation, reg_user, reg_item, reg_brand):
    raise NotImplementedError("write your pallas kernel here")


# speedup vs baseline: 7.8560x; 7.8560x over previous
import functools

import jax
import jax.numpy as jnp
from jax import lax
from jax.experimental import pallas as pl
from jax.experimental.pallas import tpu as pltpu
from jax.experimental.pallas import tpu_sc as plsc

B = 4096
D = 128


def _sc_gather(entity_re, entity_im, relation_re, relation_im,
               head, tail, relation, reg_user, reg_item, reg_brand):
    info = plsc.get_sparse_core_info()
    nw = info.num_cores * info.num_subcores
    bpw = B // nw
    mesh = plsc.VectorSubcoreMesh(core_axis_name="c", subcore_axis_name="s")
    out_t = tuple(jax.ShapeDtypeStruct((B, D), jnp.float32) for _ in range(12))

    @functools.partial(
        pl.kernel, mesh=mesh, out_type=out_t,
        scratch_types=[
            pltpu.VMEM((bpw,), jnp.int32),
            pltpu.VMEM((bpw, D), jnp.float32),
            pltpu.SemaphoreType.DMA,
        ],
    )
    def k(ent_re, ent_im, rel_re, rel_im, h, t, r, ru, ri, rb,
          o_hre, o_him, o_tre, o_tim, o_rre, o_rim,
          o_ure, o_uim, o_ire, o_iim, o_bre, o_bim,
          idx_v, rows_v, sem):
        wid = lax.axis_index("s") * info.num_cores + lax.axis_index("c")
        base = wid * bpw
        plan = [
            (h, ent_re, o_hre), (h, ent_im, o_him),
            (t, ent_re, o_tre), (t, ent_im, o_tim),
            (r, rel_re, o_rre), (r, rel_im, o_rim),
            (ru, ent_re, o_ure), (ru, ent_im, o_uim),
            (ri, ent_re, o_ire), (ri, ent_im, o_iim),
            (rb, ent_re, o_bre), (rb, ent_im, o_bim),
        ]
        prev_idx = None
        for idx_hbm, tab, out in plan:
            if idx_hbm is not prev_idx:
                pltpu.sync_copy(idx_hbm.at[pl.ds(base, bpw)], idx_v)
                prev_idx = idx_hbm
            pltpu.async_copy(tab.at[idx_v], rows_v, sem).wait()
            pltpu.sync_copy(rows_v, out.at[pl.ds(base, bpw)])

    return k(entity_re, entity_im, relation_re, relation_im,
             head, tail, relation, reg_user, reg_item, reg_brand)


def _tc_body(hre, him, tre, tim, rre, rim,
             ure, uim, ire, iim, bre, bim, score_ref, reg_ref):
    rre_ = rre[...]
    rim_ = rim[...]
    t1 = rre_ * tre[...] + rim_ * tim[...]
    t2 = rre_ * tim[...] - rim_ * tre[...]
    s = jnp.sum(hre[...] * t1 + him[...] * t2, axis=1)
    score_ref[...] = jax.nn.sigmoid(s)

    def gram_norm(a_ref, b_ref):
        a = a_ref[...]
        b = b_ref[...]
        dn = (((0,), (0,)), ((), ()))
        g = (lax.dot_general(a, a, dn, preferred_element_type=jnp.float32)
             + lax.dot_general(b, b, dn, preferred_element_type=jnp.float32))
        return jnp.sqrt(jnp.sum(g * g))

    reg = gram_norm(ure, uim) + gram_norm(ire, iim) + gram_norm(bre, bim)
    reg_ref[...] = reg.reshape(1, 1)


def kernel(entity_re, entity_im, relation_re, relation_im,
           head, tail, relation, reg_user, reg_item, reg_brand):
    gathered = _sc_gather(entity_re, entity_im, relation_re, relation_im,
                          head, tail, relation, reg_user, reg_item, reg_brand)
    score, reg = pl.pallas_call(
        _tc_body,
        out_shape=(jax.ShapeDtypeStruct((B,), jnp.float32),
                   jax.ShapeDtypeStruct((1, 1), jnp.float32)),
    )(*gathered)
    return score, reg[0, 0]


# speedup vs baseline: 9.7792x; 1.2448x over previous
import functools

import jax
import jax.numpy as jnp
from jax import lax
from jax.experimental import pallas as pl
from jax.experimental.pallas import tpu as pltpu
from jax.experimental.pallas import tpu_sc as plsc

B = 4096
D = 128


def _sc_gather(entity_re, entity_im, relation_re, relation_im,
               head, tail, relation, reg_user, reg_item, reg_brand):
    info = plsc.get_sparse_core_info()
    nw = info.num_cores * info.num_subcores
    bpw = B // nw
    nbuf = 6
    mesh = plsc.VectorSubcoreMesh(core_axis_name="c", subcore_axis_name="s")
    out_t = tuple(jax.ShapeDtypeStruct((B, D), jnp.float32) for _ in range(12))

    idx_all = jnp.stack([head, tail, relation, reg_user, reg_item, reg_brand])
    idx_all = idx_all.reshape(6, nw, bpw).transpose(1, 0, 2)

    @functools.partial(
        pl.kernel, mesh=mesh, out_type=out_t,
        scratch_types=[
            pltpu.VMEM((6, bpw), jnp.int32),
            pltpu.VMEM((nbuf, bpw, D), jnp.float32),
            pltpu.SemaphoreType.DMA((nbuf,)),
            pltpu.SemaphoreType.DMA((nbuf,)),
        ],
    )
    def k(ent_re, ent_im, rel_re, rel_im, idx_hbm,
          o_hre, o_him, o_tre, o_tim, o_rre, o_rim,
          o_ure, o_uim, o_ire, o_iim, o_bre, o_bim,
          idx_v, rows, gsem, ssem):
        wid = lax.axis_index("s") * info.num_cores + lax.axis_index("c")
        base = wid * bpw
        pltpu.sync_copy(idx_hbm.at[wid], idx_v)
        tasks = [
            (0, ent_re, o_hre), (0, ent_im, o_him),
            (1, ent_re, o_tre), (1, ent_im, o_tim),
            (2, rel_re, o_rre), (2, rel_im, o_rim),
            (3, ent_re, o_ure), (3, ent_im, o_uim),
            (4, ent_re, o_ire), (4, ent_im, o_iim),
            (5, ent_re, o_bre), (5, ent_im, o_bim),
        ]
        nt = len(tasks)
        g = [None] * nt
        s = [None] * nt

        def launch_scatter(kk):
            _, _, out = tasks[kk]
            b = kk % nbuf
            g[kk].wait()
            s[kk] = pltpu.async_copy(
                rows.at[b], out.at[pl.ds(base, bpw)], ssem.at[b])

        for t in range(nt):
            b = t % nbuf
            if t >= nbuf:
                s[t - nbuf].wait()
            j, tab, _ = tasks[t]
            g[t] = pltpu.async_copy(tab.at[idx_v.at[j]], rows.at[b],
                                    gsem.at[b])
            if t >= nbuf - 1:
                launch_scatter(t - (nbuf - 1))
        for kk in range(nt - (nbuf - 1), nt):
            launch_scatter(kk)
        for kk in range(nt - nbuf, nt):
            s[kk].wait()

    return k(entity_re, entity_im, relation_re, relation_im, idx_all)


def _tc_body(hre, him, tre, tim, rre, rim,
             ure, uim, ire, iim, bre, bim, score_ref, reg_ref):
    rre_ = rre[...]
    rim_ = rim[...]
    t1 = rre_ * tre[...] + rim_ * tim[...]
    t2 = rre_ * tim[...] - rim_ * tre[...]
    s = jnp.sum(hre[...] * t1 + him[...] * t2, axis=1)
    score_ref[...] = jax.nn.sigmoid(s)

    def gram_norm(a_ref, b_ref):
        a = a_ref[...]
        b = b_ref[...]
        dn = (((0,), (0,)), ((), ()))
        g = (lax.dot_general(a, a, dn, preferred_element_type=jnp.float32)
             + lax.dot_general(b, b, dn, preferred_element_type=jnp.float32))
        return jnp.sqrt(jnp.sum(g * g))

    reg = gram_norm(ure, uim) + gram_norm(ire, iim) + gram_norm(bre, bim)
    reg_ref[...] = reg.reshape(1, 1)


def kernel(entity_re, entity_im, relation_re, relation_im,
           head, tail, relation, reg_user, reg_item, reg_brand):
    gathered = _sc_gather(entity_re, entity_im, relation_re, relation_im,
                          head, tail, relation, reg_user, reg_item, reg_brand)
    score, reg = pl.pallas_call(
        _tc_body,
        out_shape=(jax.ShapeDtypeStruct((B,), jnp.float32),
                   jax.ShapeDtypeStruct((1, 1), jnp.float32)),
    )(*gathered)
    return score, reg[0, 0]
